# baseline (speedup 1.0000x reference)
import jax
import jax.numpy as jnp
from jax import lax
from jax.experimental import pallas as pl
from jax.experimental.pallas import tpu as pltpu
from jax.experimental.pallas import tpu_sc as plsc

TOTAL_IN = 2099200
DISC = 1024
NROWS = 2048
NCOLS = 2048
LANES = 16
NWORKERS = 32
ROWS_PER_W = NROWS // NWORKERS
G = 8
NG = ROWS_PER_W // G
NBLK = 16
WIN = 16368
WIN_TIERS = (4208, 8304, 12400, 16368)


def _tri(x):
    return (x * (x + 1)) >> 1


def _sc_body(src_hbm, disc_hbm, multi_hbm, in_buf, out_buf,
             in_sem0, in_sem1, out_sem0, out_sem1, disc_sem):
    in_sems = (in_sem0, in_sem1)
    out_sems = (out_sem0, out_sem1)
    c = lax.axis_index("c")
    s = lax.axis_index("s")
    wid = c * 16 + s
    iota = lax.iota(jnp.int32, LANES)
    neg_inf = jnp.full((LANES,), -jnp.inf, dtype=jnp.float32)

    def disc_dma():
        return pltpu.make_async_copy(
            src_hbm.at[pl.ds(0, DISC)], disc_hbm.at[0], disc_sem)

    @pl.when(wid == 0)
    def _():
        disc_dma().start()

    def gbase(g):
        rg = (wid + NWORKERS * g) * G
        startg = DISC + _tri(rg)
        a = jnp.minimum((startg >> 3) << 3, TOTAL_IN - WIN)
        a = pl.multiple_of(a, 8)
        return rg, a

    def in_dma(g, b, win):
        _, a = gbase(g)
        return pltpu.make_async_copy(
            src_hbm.at[pl.ds(a, win)],
            in_buf.at[pl.ds(b * WIN, win)], in_sems[b])

    def out_dma(g, b, half):
        rg, _ = gbase(g)
        h = G // 2
        return pltpu.make_async_copy(
            out_buf.at[b, pl.ds(half * h, h)],
            multi_hbm.at[pl.ds(rg + half * h, h)], out_sems[b])

    def wait_out(g, b):
        out_dma(g, b, 0).wait()
        out_dma(g, b, 1).wait()

    def start_in(g, b):
        tier = g >> 1
        for i, w in enumerate(WIN_TIERS):
            @pl.when(tier == i)
            def _(g=g, b=b, w=w):
                in_dma(g, b, w).start()

    def wait_in(g, b):
        tier = g >> 1
        for i, w in enumerate(WIN_TIERS):
            @pl.when(tier == i)
            def _(g=g, b=b, w=w):
                in_dma(g, b, w).wait()

    start_in(0, 0)

    @pl.loop(0, NG // 2)
    def _(gg):
        for b in range(2):
            g = gg * 2 + b

            @pl.when(g + 1 < NG)
            def _():
                start_in(g + 1, 1 - b)

            wait_in(g, b)

            @pl.when(g >= 2)
            def _():
                wait_out(g - 2, b)

            rg, a = gbase(g)
            ibase = b * WIN

            def assemble(t, rg=rg, a=a, ibase=ibase, b=b):
                r = rg + t
                off = ibase + DISC + _tri(r) - a
                nvalid = r + 1
                bblk = jnp.minimum((nvalid >> 4) >> 3, NBLK - 1)
                ob = out_buf.at[b, t]

                @plsc.parallel_loop(0, bblk, unroll=4)
                def _(blk, off=off, ob=ob):
                    for jj in range(8):
                        cw = (blk * 8 + jj) * LANES
                        ob[pl.ds(pl.multiple_of(cw, LANES), LANES)] = (
                            in_buf[pl.ds(off + cw, LANES)])

                for jj in range(8):
                    cw = (bblk * 8 + jj) * LANES
                    data = in_buf[pl.ds(off + cw, LANES)]
                    ob[pl.ds(pl.multiple_of(cw, LANES), LANES)] = jnp.where(
                        iota + cw < nvalid, data, neg_inf)

                @pl.when(gg == 0)
                def _():
                    @plsc.parallel_loop(bblk + 1, NBLK, unroll=2)
                    def _(blk, ob=ob):
                        for jj in range(8):
                            cw = (blk * 8 + jj) * LANES
                            ob[pl.ds(pl.multiple_of(cw, LANES),
                                     LANES)] = neg_inf

            pl.loop(0, G // 2)(assemble)
            out_dma(g, b, 0).start()
            pl.loop(G // 2, G)(assemble)
            out_dma(g, b, 1).start()

    wait_out(NG - 2, 0)
    wait_out(NG - 1, 1)

    @pl.when(wid == 0)
    def _():
        disc_dma().wait()


def kernel(logits):
    mesh = plsc.VectorSubcoreMesh(core_axis_name="c", subcore_axis_name="s")
    out_type = (
        jax.ShapeDtypeStruct((1, DISC), jnp.float32),
        jax.ShapeDtypeStruct((NROWS, NCOLS), jnp.float32),
    )
    f = pl.kernel(
        _sc_body,
        out_type=out_type,
        mesh=mesh,
        scratch_types=[
            pltpu.VMEM((2 * WIN,), jnp.float32),
            pltpu.VMEM((2, G, NCOLS), jnp.float32),
            pltpu.SemaphoreType.DMA,
            pltpu.SemaphoreType.DMA,
            pltpu.SemaphoreType.DMA,
            pltpu.SemaphoreType.DMA,
            pltpu.SemaphoreType.DMA,
        ],
    )
    disc, multi = f(logits)
    return {"disc": disc, "multi": multi}


# speedup vs baseline: 1.0390x; 1.0390x over previous
import jax
import jax.numpy as jnp
from jax import lax
from jax.experimental import pallas as pl
from jax.experimental.pallas import tpu as pltpu
from jax.experimental.pallas import tpu_sc as plsc

TOTAL_IN = 2099200
DISC = 1024
NROWS = 2048
NCOLS = 2048
LANES = 16
NWORKERS = 32
ROWS_PER_W = NROWS // NWORKERS
G = 8
NG = ROWS_PER_W // G
NBLK = 16
WIN = 16368
WIN_TIERS = (4208, 8304, 12400, 16368)


def _tri(x):
    return (x * (x + 1)) >> 1


def _sc_body(src_hbm, disc_hbm, multi_hbm, in_buf, out_buf,
             in_sem0, in_sem1, out_sem0, out_sem1, disc_sem):
    in_sems = (in_sem0, in_sem1)
    out_sems = (out_sem0, out_sem1)
    c = lax.axis_index("c")
    s = lax.axis_index("s")
    wid = c * 16 + s
    iota = lax.iota(jnp.int32, LANES)
    neg_inf = jnp.full((LANES,), -jnp.inf, dtype=jnp.float32)

    def disc_dma():
        return pltpu.make_async_copy(
            src_hbm.at[pl.ds(0, DISC)], disc_hbm.at[0], disc_sem)

    @pl.when(wid == 0)
    def _():
        disc_dma().start()

    def gbase(g):
        rg = (wid + NWORKERS * g) * G
        startg = DISC + _tri(rg)
        a = jnp.minimum((startg >> 3) << 3, TOTAL_IN - WIN)
        a = pl.multiple_of(a, 8)
        return rg, a

    def in_dma(g, b, win):
        _, a = gbase(g)
        return pltpu.make_async_copy(
            src_hbm.at[pl.ds(a, win)],
            in_buf.at[pl.ds(b * WIN, win)], in_sems[b])

    def out_dma(g, b):
        rg, _ = gbase(g)
        return pltpu.make_async_copy(
            out_buf.at[b], multi_hbm.at[pl.ds(rg, G)], out_sems[b])

    def wait_out(g, b):
        out_dma(g, b).wait()

    def start_in(g, b):
        tier = g >> 1
        for i, w in enumerate(WIN_TIERS):
            @pl.when(tier == i)
            def _(g=g, b=b, w=w):
                in_dma(g, b, w).start()

    def wait_in(g, b):
        tier = g >> 1
        for i, w in enumerate(WIN_TIERS):
            @pl.when(tier == i)
            def _(g=g, b=b, w=w):
                in_dma(g, b, w).wait()

    start_in(0, 0)

    @pl.loop(0, NG // 2)
    def _(gg):
        for b in range(2):
            g = gg * 2 + b

            @pl.when(g + 1 < NG)
            def _():
                start_in(g + 1, 1 - b)

            wait_in(g, b)

            @pl.when(g >= 2)
            def _():
                wait_out(g - 2, b)

            rg, a = gbase(g)
            ibase = b * WIN

            def assemble(t, rg=rg, a=a, ibase=ibase, b=b):
                r = rg + t
                off = ibase + DISC + _tri(r) - a
                nvalid = r + 1
                bblk = jnp.minimum((nvalid >> 4) >> 3, NBLK - 1)
                ob = out_buf.at[b, t]

                @plsc.parallel_loop(0, bblk, unroll=4)
                def _(blk, off=off, ob=ob):
                    for jj in range(8):
                        cw = (blk * 8 + jj) * LANES
                        ob[pl.ds(pl.multiple_of(cw, LANES), LANES)] = (
                            in_buf[pl.ds(off + cw, LANES)])

                for jj in range(8):
                    cw = (bblk * 8 + jj) * LANES
                    data = in_buf[pl.ds(off + cw, LANES)]
                    ob[pl.ds(pl.multiple_of(cw, LANES), LANES)] = jnp.where(
                        iota + cw < nvalid, data, neg_inf)

                @pl.when(gg == 0)
                def _():
                    @plsc.parallel_loop(bblk + 1, NBLK, unroll=2)
                    def _(blk, ob=ob):
                        for jj in range(8):
                            cw = (blk * 8 + jj) * LANES
                            ob[pl.ds(pl.multiple_of(cw, LANES),
                                     LANES)] = neg_inf

            pl.loop(0, G)(assemble)
            out_dma(g, b).start()

    wait_out(NG - 2, 0)
    wait_out(NG - 1, 1)

    @pl.when(wid == 0)
    def _():
        disc_dma().wait()


def kernel(logits):
    mesh = plsc.VectorSubcoreMesh(core_axis_name="c", subcore_axis_name="s")
    out_type = (
        jax.ShapeDtypeStruct((1, DISC), jnp.float32),
        jax.ShapeDtypeStruct((NROWS, NCOLS), jnp.float32),
    )
    f = pl.kernel(
        _sc_body,
        out_type=out_type,
        mesh=mesh,
        scratch_types=[
            pltpu.VMEM((2 * WIN,), jnp.float32),
            pltpu.VMEM((2, G, NCOLS), jnp.float32),
            pltpu.SemaphoreType.DMA,
            pltpu.SemaphoreType.DMA,
            pltpu.SemaphoreType.DMA,
            pltpu.SemaphoreType.DMA,
            pltpu.SemaphoreType.DMA,
        ],
    )
    disc, multi = f(logits)
    return {"disc": disc, "multi": multi}
